# baseline (device time: 16597 ns/iter reference)
import jax
import jax.numpy as jnp
from jax import lax
from jax.experimental import pallas as pl
from jax.experimental.pallas import tpu as pltpu

N_DEV = 16
SQ = 512
D = 1024
DH = 128
HQ_LOCAL = 8
CH = SQ // N_DEV
SCALE = 0.08838834764831843


def kernel(x, Wq, Wo, Wk, Wv):
    idx = lax.axis_index("i")
    x2 = x.reshape(SQ, D)
    wk_l = lax.dynamic_slice(Wk, (0, idx * (2 * DH)), (D, 2 * DH))
    wv_l = lax.dynamic_slice(Wv, (0, idx * (2 * DH)), (D, 2 * DH))

    def body(x_ref, wq_ref, wk_ref, wv_ref, wo_ref, out_ref,
             acc_ref, rs_buf, ag_buf, rs_send, rs_recv, ag_send, ag_recv):
        my = lax.axis_index("i")
        left = lax.rem(my + N_DEV - 1, N_DEV)
        right = lax.rem(my + 1, N_DEV)

        xv = x_ref[:]
        q = jnp.dot(xv, wq_ref[:], preferred_element_type=jnp.float32)
        k = jnp.dot(xv, wk_ref[:], preferred_element_type=jnp.float32)
        v = jnp.dot(xv, wv_ref[:], preferred_element_type=jnp.float32)
        outs = []
        for h in range(HQ_LOCAL):
            g = h // 4
            qh = q[:, h * DH:(h + 1) * DH]
            kg = k[:, g * DH:(g + 1) * DH]
            vg = v[:, g * DH:(g + 1) * DH]
            s = lax.dot_general(qh, kg, (((1,), (1,)), ((), ())),
                                preferred_element_type=jnp.float32) * SCALE
            m = jnp.max(s, axis=-1, keepdims=True)
            p = jnp.exp(s - m)
            l = jnp.sum(p, axis=-1, keepdims=True)
            outs.append(jnp.dot(p, vg, preferred_element_type=jnp.float32) / l)
        a = jnp.concatenate(outs, axis=1)
        acc_ref[:] = jnp.dot(a, wo_ref[:], preferred_element_type=jnp.float32)

        bar = pltpu.get_barrier_semaphore()
        for nbr in (left, right):
            pl.semaphore_signal(bar, inc=1, device_id=(nbr,),
                                device_id_type=pl.DeviceIdType.MESH)
        pl.semaphore_wait(bar, 2)

        for st in range(N_DEV - 1):
            send_chunk = lax.rem(my + N_DEV - st, N_DEV)
            if st == 0:
                src = acc_ref.at[pl.ds(send_chunk * CH, CH)]
            else:
                rs_buf[st - 1, :, :] = (rs_buf[st - 1, :, :]
                                        + acc_ref[pl.ds(send_chunk * CH, CH)])
                src = rs_buf.at[st - 1]
            rdma = pltpu.make_async_remote_copy(
                src_ref=src,
                dst_ref=rs_buf.at[st],
                send_sem=rs_send.at[st],
                recv_sem=rs_recv.at[st],
                device_id=(right,),
                device_id_type=pl.DeviceIdType.MESH,
            )
            rdma.start()
            rdma.wait()

        red_chunk = right
        out_ref[pl.ds(red_chunk * CH, CH), :] = (
            rs_buf[N_DEV - 2, :, :] + acc_ref[pl.ds(red_chunk * CH, CH)])

        for hp in range(N_DEV - 1):
            send_chunk = lax.rem(my + 1 + N_DEV - hp, N_DEV)
            rdma = pltpu.make_async_remote_copy(
                src_ref=out_ref.at[pl.ds(send_chunk * CH, CH)],
                dst_ref=ag_buf.at[hp],
                send_sem=ag_send.at[hp],
                recv_sem=ag_recv.at[hp],
                device_id=(right,),
                device_id_type=pl.DeviceIdType.MESH,
            )
            rdma.start()
            rdma.wait()
            recv_chunk = lax.rem(my + N_DEV - hp, N_DEV)
            out_ref[pl.ds(recv_chunk * CH, CH), :] = ag_buf[hp, :, :]

    out = pl.pallas_call(
        body,
        out_shape=jax.ShapeDtypeStruct((SQ, D), jnp.float32),
        in_specs=[pl.BlockSpec(memory_space=pltpu.VMEM)] * 5,
        out_specs=pl.BlockSpec(memory_space=pltpu.VMEM),
        scratch_shapes=[
            pltpu.VMEM((SQ, D), jnp.float32),
            pltpu.VMEM((N_DEV - 1, CH, D), jnp.float32),
            pltpu.VMEM((N_DEV - 1, CH, D), jnp.float32),
            pltpu.SemaphoreType.DMA((N_DEV - 1,)),
            pltpu.SemaphoreType.DMA((N_DEV - 1,)),
            pltpu.SemaphoreType.DMA((N_DEV - 1,)),
            pltpu.SemaphoreType.DMA((N_DEV - 1,)),
        ],
        compiler_params=pltpu.CompilerParams(collective_id=0),
    )(x2, Wq, wk_l, wv_l, Wo)
    return out.reshape(1, SQ, D)
